# baseline (speedup 1.0000x reference)
import functools

import jax
import jax.numpy as jnp
from jax import lax
from jax.experimental import pallas as pl
from jax.experimental.pallas import tpu as pltpu
from jax.experimental.pallas import tpu_sc as plsc

_B = 4096
_S = 200
_D = 64

_NC = 2
_NS = 16
_NW = _NC * _NS
_BLK = _B // _NW
_NBUF = 4

_mesh = plsc.VectorSubcoreMesh(core_axis_name="c", subcore_axis_name="s")


@functools.partial(
    pl.kernel,
    out_type=jax.ShapeDtypeStruct((_S, _D, _B), jnp.float32),
    mesh=_mesh,
    scratch_types=[
        pltpu.VMEM((_S, _BLK), jnp.int32),
        pltpu.VMEM((_NBUF, _BLK, _D), jnp.float32),
        pltpu.VMEM((_NBUF, _D, _BLK), jnp.float32),
        pltpu.SemaphoreType.DMA,
        pltpu.SemaphoreType.DMA,
        pltpu.SemaphoreType.DMA,
        pltpu.SemaphoreType.DMA,
        pltpu.SemaphoreType.DMA,
        pltpu.SemaphoreType.DMA,
        pltpu.SemaphoreType.DMA,
        pltpu.SemaphoreType.DMA,
    ],
    compiler_params=pltpu.CompilerParams(
        use_tc_tiling_on_sc=False, needs_layout_passes=False
    ),
)
def _sc_gather_t(xt_hbm, table_hbm, out_hbm, idx_v, rows_v, tblk_v,
                 g0, g1, g2, g3, o0, o1, o2, o3):
    w = lax.axis_index("s") * _NC + lax.axis_index("c")
    b0 = w * _BLK
    col = lax.iota(jnp.int32, 16)
    rowidx = [16 * j + col for j in range(4)]
    gsems = (g0, g1, g2, g3)
    osems = (o0, o1, o2, o3)

    pltpu.sync_copy(xt_hbm.at[:, pl.ds(b0, _BLK)], idx_v)

    def gather(s, p):
        return pltpu.make_async_copy(
            table_hbm.at[idx_v.at[s]], rows_v.at[p], gsems[p]
        )

    def transpose(p):
        rows = rows_v.at[p]
        tdst = tblk_v.at[p]

        @plsc.parallel_loop(0, _BLK, unroll=8)
        def row(b):
            bvec = jnp.full((16,), b, jnp.int32)
            for j in range(4):
                v = rows[b, pl.ds(16 * j, 16)]
                plsc.store_scatter(tdst, [rowidx[j], bvec], v)

    def out_copy(s, p):
        return pltpu.make_async_copy(
            tblk_v.at[p], out_hbm.at[s, :, pl.ds(b0, _BLK)], osems[p]
        )

    for s in range(_NBUF - 1):
        gather(s, s).start()

    def ring(i, carry):
        s0 = _NBUF * i
        for p in range(_NBUF):
            s = s0 + p

            @pl.when(s + _NBUF - 1 < _S)
            def _():
                gather(s + _NBUF - 1, (p + _NBUF - 1) % _NBUF).start()

            gather(s, p).wait()

            @pl.when(s >= _NBUF)
            def _():
                out_copy(s - _NBUF, p).wait()

            transpose(p)
            out_copy(s, p).start()
        return carry

    lax.fori_loop(0, _S // _NBUF, ring, 0)
    for s in range(_S - _NBUF, _S):
        out_copy(s, s % _NBUF).wait()


def kernel(x, vocab_embed):
    b, s = x.shape
    d = vocab_embed.shape[1]
    xt = x.T
    outt = _sc_gather_t(xt, vocab_embed)
    real = outt.transpose(2, 0, 1)
    omega = 1.0 / (10000.0 ** (jnp.arange(0, d, 2, dtype=jnp.float32) / d))
    angles = omega[None, :] * jnp.arange(s, dtype=jnp.float32)[:, None]
    imag = jnp.repeat(jnp.sin(angles), 2, axis=-1)
    imag = jnp.broadcast_to(imag[None, :, :], (b, s, d))
    return jax.lax.complex(real, imag)
